# baseline (speedup 1.0000x reference)
import functools

import jax
import jax.numpy as jnp
from jax import lax
from jax.experimental import pallas as pl
from jax.experimental.pallas import tpu as pltpu
from jax.experimental.pallas import tpu_sc as plsc

NC = 2
NS = 16
NW = NC * NS

ROWS_PER_UNIT = 1
NBUF = 8


def _make_pool_kernel(B, S, D, SPAD):
    RW = B // NW
    U = RW // ROWS_PER_UNIT
    IPU = ROWS_PER_UNIT * S
    assert IPU <= 128
    ND = D // 16

    mesh = plsc.VectorSubcoreMesh(
        core_axis_name="c", subcore_axis_name="s", num_cores=NC, num_subcores=NS)

    @functools.partial(
        pl.kernel,
        out_type=jax.ShapeDtypeStruct((B, D), jnp.float32),
        mesh=mesh,
        scratch_types=[
            pltpu.VMEM((U, SPAD), jnp.int32),
            pltpu.VMEM((NBUF, IPU, D), jnp.float32),
            pltpu.VMEM((RW, D), jnp.float32),
            pltpu.SemaphoreType.DMA((NBUF,)),
        ],
    )
    def pool(x_hbm, table_hbm, out_hbm, idx_v, rows, out_v, sems):
        wid = lax.axis_index("s") * NC + lax.axis_index("c")

        pltpu.sync_copy(x_hbm.at[pl.ds(wid * U, U)], idx_v)

        def start(unit, b):
            pltpu.async_copy(
                table_hbm.at[idx_v.at[unit, pl.ds(0, IPU)]], rows.at[b],
                sems.at[b])

        def wait(b):
            pltpu.make_async_copy(
                table_hbm.at[idx_v.at[0, pl.ds(0, IPU)]], rows.at[b],
                sems.at[b]).wait()

        def accumulate(unit, b):
            buf = rows.at[b]
            for r in range(ROWS_PER_UNIT):
                def body(s, acc):
                    return tuple(
                        acc[d] + buf[r * S + s, pl.ds(d * 16, 16)]
                        for d in range(ND)
                    )
                acc = lax.fori_loop(
                    0, S, body,
                    tuple(jnp.zeros((16,), jnp.float32) for _ in range(ND)),
                )
                row = unit * ROWS_PER_UNIT + r
                for d in range(ND):
                    out_v[row, pl.ds(d * 16, 16)] = acc[d]

        @pl.loop(0, NBUF)
        def _(g):
            start(g, g)

        @pl.loop(0, U - NBUF)
        def _(g):
            b = lax.rem(g, NBUF)
            wait(b)
            accumulate(g, b)
            start(g + NBUF, b)

        @pl.loop(U - NBUF, U)
        def _(g):
            b = lax.rem(g, NBUF)
            wait(b)
            accumulate(g, b)

        pltpu.sync_copy(out_v, out_hbm.at[pl.ds(wid * RW, RW)])

    return pool


def _mlp_block(h_ref, w1_ref, b1_ref, w2_ref, b2_ref, w3_ref, b3_ref, o_ref):
    h = h_ref[...]
    h = jnp.maximum(jnp.dot(h, w1_ref[...], preferred_element_type=jnp.float32)
                    + b1_ref[...], 0.0)
    h = jnp.maximum(jnp.dot(h, w2_ref[...], preferred_element_type=jnp.float32)
                    + b2_ref[...], 0.0)
    h = jnp.maximum(jnp.dot(h, w2_ref[...], preferred_element_type=jnp.float32)
                    + b2_ref[...], 0.0)
    o_ref[...] = (jnp.dot(h, w3_ref[...], preferred_element_type=jnp.float32)
                  + b3_ref[...])


def _mlp(pooled, W1s, b1, W2, b2, W3, b3):
    B, D = pooled.shape
    HID = W2.shape[0]
    ALTS = W3.shape[1]
    BT = 2048
    grid = (B // BT,)
    full = lambda shape: pl.BlockSpec(shape, lambda i: (0, 0))
    return pl.pallas_call(
        _mlp_block,
        grid=grid,
        in_specs=[
            pl.BlockSpec((BT, D), lambda i: (i, 0)),
            full((D, HID)), full((1, HID)),
            full((HID, HID)), full((1, HID)),
            full((HID, ALTS)), full((1, ALTS)),
        ],
        out_specs=pl.BlockSpec((BT, ALTS), lambda i: (i, 0)),
        out_shape=jax.ShapeDtypeStruct((B, ALTS), jnp.float32),
    )(pooled, W1s, b1.reshape(1, HID), W2, b2.reshape(1, HID),
      W3, b3.reshape(1, ALTS))


@jax.jit
def kernel(x, table, W1, b1, W2, b2, W3, b3):
    B, S = x.shape
    V, D = table.shape
    SPAD = 128
    x2 = jnp.pad(x.astype(jnp.int32), ((0, 0), (0, SPAD - S)))
    W1s = W1 * (1.0 / S)
    pooled = _make_pool_kernel(B, S, D, SPAD)(x2, table)
    return _mlp(pooled, W1s, b1, W2, b2, W3, b3)


# speedup vs baseline: 1.0019x; 1.0019x over previous
import functools

import jax
import jax.numpy as jnp
from jax import lax
from jax.experimental import pallas as pl
from jax.experimental.pallas import tpu as pltpu
from jax.experimental.pallas import tpu_sc as plsc

NC = 2
NS = 16
NW = NC * NS

ROWS_PER_UNIT = 1
NBUF = 8


def _make_pool_kernel(B, S, D, SPAD):
    RW = B // NW
    U = RW // ROWS_PER_UNIT
    IPU = ROWS_PER_UNIT * S
    assert IPU <= 128
    ND = D // 16

    mesh = plsc.VectorSubcoreMesh(
        core_axis_name="c", subcore_axis_name="s", num_cores=NC, num_subcores=NS)

    @functools.partial(
        pl.kernel,
        out_type=jax.ShapeDtypeStruct((B, D), jnp.float32),
        mesh=mesh,
        scratch_types=[
            pltpu.VMEM((U * SPAD,), jnp.int32),
            pltpu.VMEM((NBUF, IPU, D), jnp.float32),
            pltpu.VMEM((RW, D), jnp.float32),
            pltpu.SemaphoreType.DMA((NBUF,)),
        ],
    )
    def pool(x_hbm, table_hbm, out_hbm, idx_v, rows, out_v, sems):
        wid = lax.axis_index("s") * NC + lax.axis_index("c")

        pltpu.sync_copy(x_hbm.at[pl.ds(wid * U * SPAD, U * SPAD)], idx_v)

        def start(unit, b):
            pltpu.async_copy(
                table_hbm.at[idx_v.at[pl.ds(unit * SPAD, IPU)]], rows.at[b],
                sems.at[b])

        def wait(b):
            pltpu.make_async_copy(
                table_hbm.at[idx_v.at[pl.ds(0, IPU)]], rows.at[b],
                sems.at[b]).wait()

        def accumulate(unit, b):
            buf = rows.at[b]
            for r in range(ROWS_PER_UNIT):
                def body(s, acc):
                    return tuple(
                        acc[d] + buf[r * S + s, pl.ds(d * 16, 16)]
                        for d in range(ND)
                    )
                acc = lax.fori_loop(
                    0, S, body,
                    tuple(jnp.zeros((16,), jnp.float32) for _ in range(ND)),
                )
                row = unit * ROWS_PER_UNIT + r
                for d in range(ND):
                    out_v[row, pl.ds(d * 16, 16)] = acc[d]

        @pl.loop(0, NBUF)
        def _(g):
            start(g, g)

        @pl.loop(0, U - NBUF)
        def _(g):
            b = lax.rem(g, NBUF)
            wait(b)
            accumulate(g, b)
            start(g + NBUF, b)

        @pl.loop(U - NBUF, U)
        def _(g):
            b = lax.rem(g, NBUF)
            wait(b)
            accumulate(g, b)

        pltpu.sync_copy(out_v, out_hbm.at[pl.ds(wid * RW, RW)])

    return pool


def _mlp_block(h_ref, w1_ref, b1_ref, w2_ref, b2_ref, w3_ref, b3_ref, o_ref):
    h = h_ref[...]
    h = jnp.maximum(jnp.dot(h, w1_ref[...], preferred_element_type=jnp.float32)
                    + b1_ref[...], 0.0)
    h = jnp.maximum(jnp.dot(h, w2_ref[...], preferred_element_type=jnp.float32)
                    + b2_ref[...], 0.0)
    h = jnp.maximum(jnp.dot(h, w2_ref[...], preferred_element_type=jnp.float32)
                    + b2_ref[...], 0.0)
    o_ref[...] = (jnp.dot(h, w3_ref[...], preferred_element_type=jnp.float32)
                  + b3_ref[...])


def _mlp(pooled, W1s, b1, W2, b2, W3, b3):
    B, D = pooled.shape
    HID = W2.shape[0]
    ALTS = W3.shape[1]
    BT = 2048
    grid = (B // BT,)
    full = lambda shape: pl.BlockSpec(shape, lambda i: (0, 0))
    return pl.pallas_call(
        _mlp_block,
        grid=grid,
        in_specs=[
            pl.BlockSpec((BT, D), lambda i: (i, 0)),
            full((D, HID)), full((1, HID)),
            full((HID, HID)), full((1, HID)),
            full((HID, ALTS)), full((1, ALTS)),
        ],
        out_specs=pl.BlockSpec((BT, ALTS), lambda i: (i, 0)),
        out_shape=jax.ShapeDtypeStruct((B, ALTS), jnp.float32),
    )(pooled, W1s, b1.reshape(1, HID), W2, b2.reshape(1, HID),
      W3, b3.reshape(1, ALTS))


@jax.jit
def kernel(x, table, W1, b1, W2, b2, W3, b3):
    B, S = x.shape
    V, D = table.shape
    SPAD = 128
    x2 = jnp.pad(x.astype(jnp.int32), ((0, 0), (0, SPAD - S))).reshape(-1)
    W1s = W1 * (1.0 / S)
    pooled = _make_pool_kernel(B, S, D, SPAD)(x2, table)
    return _mlp(pooled, W1s, b1, W2, b2, W3, b3)


# speedup vs baseline: 1.0041x; 1.0022x over previous
import functools

import jax
import jax.numpy as jnp
from jax import lax
from jax.experimental import pallas as pl
from jax.experimental.pallas import tpu as pltpu
from jax.experimental.pallas import tpu_sc as plsc

NC = 2
NS = 16
NW = NC * NS

ROWS_PER_UNIT = 1
NBUF = 8


def _make_pool_kernel(B, S, D, SPAD):
    RW = B // NW
    U = RW // ROWS_PER_UNIT
    IPU = ROWS_PER_UNIT * S
    assert IPU <= 128
    ND = D // 16

    mesh = plsc.VectorSubcoreMesh(
        core_axis_name="c", subcore_axis_name="s", num_cores=NC, num_subcores=NS)

    @functools.partial(
        pl.kernel,
        out_type=jax.ShapeDtypeStruct((B, D), jnp.float32),
        mesh=mesh,
        scratch_types=[
            pltpu.VMEM((U * SPAD,), jnp.int32),
            pltpu.VMEM((NBUF, IPU, D), jnp.float32),
            pltpu.VMEM((RW, D), jnp.float32),
            pltpu.SemaphoreType.DMA((NBUF,)),
        ],
    )
    def pool(x_hbm, table_hbm, out_hbm, idx_v, rows, out_v, sems):
        wid = lax.axis_index("s") * NC + lax.axis_index("c")

        pltpu.sync_copy(x_hbm.at[pl.ds(wid * U * SPAD, U * SPAD)], idx_v)

        def start(unit, b):
            pltpu.async_copy(
                table_hbm.at[idx_v.at[pl.ds(unit * SPAD, IPU)]], rows.at[b],
                sems.at[b])

        def wait(b):
            pltpu.make_async_copy(
                table_hbm.at[idx_v.at[pl.ds(0, IPU)]], rows.at[b],
                sems.at[b]).wait()

        def accumulate(unit, b):
            buf = rows.at[b]
            for r in range(ROWS_PER_UNIT):
                def body(s, acc):
                    return tuple(
                        acc[d] + buf[r * S + s, pl.ds(d * 16, 16)]
                        for d in range(ND)
                    )
                acc = lax.fori_loop(
                    0, S, body,
                    tuple(jnp.zeros((16,), jnp.float32) for _ in range(ND)),
                )
                row = unit * ROWS_PER_UNIT + r
                for d in range(ND):
                    out_v[row, pl.ds(d * 16, 16)] = acc[d]

        @pl.loop(0, NBUF)
        def _(g):
            start(g, g)

        @pl.loop(0, U - NBUF)
        def _(g):
            b = lax.rem(g, NBUF)
            wait(b)
            accumulate(g, b)
            start(g + NBUF, b)

        @pl.loop(U - NBUF, U)
        def _(g):
            b = lax.rem(g, NBUF)
            wait(b)
            accumulate(g, b)

        pltpu.sync_copy(out_v, out_hbm.at[pl.ds(wid * RW, RW)])

    return pool


def _mlp_block(h_ref, w1_ref, b1_ref, w2_ref, b2_ref, w3_ref, b3_ref, o_ref):
    h = h_ref[...]
    h = jnp.maximum(jnp.dot(h, w1_ref[...], preferred_element_type=jnp.float32)
                    + b1_ref[...], 0.0)
    h = jnp.maximum(jnp.dot(h, w2_ref[...], preferred_element_type=jnp.float32)
                    + b2_ref[...], 0.0)
    h = jnp.maximum(jnp.dot(h, w2_ref[...], preferred_element_type=jnp.float32)
                    + b2_ref[...], 0.0)
    o_ref[...] = (jnp.dot(h, w3_ref[...], preferred_element_type=jnp.float32)
                  + b3_ref[...])


def _mlp(pooled, W1s, b1, W2, b2, W3, b3):
    B, D = pooled.shape
    HID = W2.shape[0]
    ALTS = W3.shape[1]
    APAD = 128
    W3 = jnp.pad(W3, ((0, 0), (0, APAD - ALTS)))
    b3 = jnp.pad(b3, ((0, APAD - ALTS),))
    BT = 2048
    grid = (B // BT,)
    full = lambda shape: pl.BlockSpec(shape, lambda i: (0, 0))
    return pl.pallas_call(
        _mlp_block,
        grid=grid,
        in_specs=[
            pl.BlockSpec((BT, D), lambda i: (i, 0)),
            full((D, HID)), full((1, HID)),
            full((HID, HID)), full((1, HID)),
            full((HID, APAD)), full((1, APAD)),
        ],
        out_specs=pl.BlockSpec((BT, APAD), lambda i: (i, 0)),
        out_shape=jax.ShapeDtypeStruct((B, APAD), jnp.float32),
    )(pooled, W1s, b1.reshape(1, HID), W2, b2.reshape(1, HID),
      W3, b3.reshape(1, APAD))[:, :ALTS]


@jax.jit
def kernel(x, table, W1, b1, W2, b2, W3, b3):
    B, S = x.shape
    V, D = table.shape
    SPAD = 128
    x2 = jnp.pad(x.astype(jnp.int32), ((0, 0), (0, SPAD - S))).reshape(-1)
    W1s = W1 * (1.0 / S)
    pooled = _make_pool_kernel(B, S, D, SPAD)(x2, table)
    return _mlp(pooled, W1s, b1, W2, b2, W3, b3)


# speedup vs baseline: 1.0056x; 1.0015x over previous
import functools

import jax
import jax.numpy as jnp
from jax import lax
from jax.experimental import pallas as pl
from jax.experimental.pallas import tpu as pltpu
from jax.experimental.pallas import tpu_sc as plsc

NC = 2
NS = 16
NW = NC * NS

ROWS_PER_UNIT = 1
NBUF = 8


def _make_pool_kernel(B, S, D):
    RW = B // NW
    U = RW // ROWS_PER_UNIT
    IPU = ROWS_PER_UNIT * S
    assert IPU <= 128
    ND = D // 16

    mesh = plsc.VectorSubcoreMesh(
        core_axis_name="c", subcore_axis_name="s", num_cores=NC, num_subcores=NS)

    @functools.partial(
        pl.kernel,
        out_type=jax.ShapeDtypeStruct((B, D), jnp.float32),
        mesh=mesh,
        scratch_types=[
            pltpu.VMEM((U, IPU), jnp.int32),
            pltpu.VMEM((NBUF, IPU, D), jnp.float32),
            pltpu.VMEM((RW, D), jnp.float32),
            pltpu.SemaphoreType.DMA((NBUF,)),
        ],
    )
    def pool(x_hbm, table_hbm, out_hbm, idx_v, rows, out_v, sems):
        wid = lax.axis_index("s") * NC + lax.axis_index("c")

        pltpu.sync_copy(x_hbm.at[pl.ds(wid * U, U)], idx_v)

        def start(unit, b):
            pltpu.async_copy(
                table_hbm.at[idx_v.at[unit]], rows.at[b], sems.at[b])

        def wait(b):
            pltpu.make_async_copy(
                table_hbm.at[idx_v.at[0]], rows.at[b], sems.at[b]).wait()

        def accumulate(unit, b):
            buf = rows.at[b]
            for r in range(ROWS_PER_UNIT):
                def body(s, acc):
                    return tuple(
                        acc[d] + buf[r * S + s, pl.ds(d * 16, 16)]
                        for d in range(ND)
                    )
                acc = lax.fori_loop(
                    0, S, body,
                    tuple(jnp.zeros((16,), jnp.float32) for _ in range(ND)),
                )
                row = unit * ROWS_PER_UNIT + r
                for d in range(ND):
                    out_v[row, pl.ds(d * 16, 16)] = acc[d]

        @pl.loop(0, NBUF)
        def _(g):
            start(g, g)

        @pl.loop(0, U - NBUF)
        def _(g):
            b = lax.rem(g, NBUF)
            wait(b)
            accumulate(g, b)
            start(g + NBUF, b)

        @pl.loop(U - NBUF, U)
        def _(g):
            b = lax.rem(g, NBUF)
            wait(b)
            accumulate(g, b)

        pltpu.sync_copy(out_v, out_hbm.at[pl.ds(wid * RW, RW)])

    return pool


def _mlp_block(h_ref, w1_ref, b1_ref, w2_ref, b2_ref, w3_ref, b3_ref, o_ref):
    h = h_ref[...]
    h = jnp.maximum(jnp.dot(h, w1_ref[...], preferred_element_type=jnp.float32)
                    + b1_ref[...], 0.0)
    h = jnp.maximum(jnp.dot(h, w2_ref[...], preferred_element_type=jnp.float32)
                    + b2_ref[...], 0.0)
    h = jnp.maximum(jnp.dot(h, w2_ref[...], preferred_element_type=jnp.float32)
                    + b2_ref[...], 0.0)
    o_ref[...] = (jnp.dot(h, w3_ref[...], preferred_element_type=jnp.float32)
                  + b3_ref[...])


def _mlp(pooled, W1s, b1, W2, b2, W3, b3):
    B, D = pooled.shape
    HID = W2.shape[0]
    ALTS = W3.shape[1]
    BT = 2048
    grid = (B // BT,)
    full = lambda shape: pl.BlockSpec(shape, lambda i: (0, 0))
    return pl.pallas_call(
        _mlp_block,
        grid=grid,
        in_specs=[
            pl.BlockSpec((BT, D), lambda i: (i, 0)),
            full((D, HID)), full((1, HID)),
            full((HID, HID)), full((1, HID)),
            full((HID, ALTS)), full((1, ALTS)),
        ],
        out_specs=pl.BlockSpec((BT, ALTS), lambda i: (i, 0)),
        out_shape=jax.ShapeDtypeStruct((B, ALTS), jnp.float32),
    )(pooled, W1s, b1.reshape(1, HID), W2, b2.reshape(1, HID),
      W3, b3.reshape(1, ALTS))


@jax.jit
def kernel(x, table, W1, b1, W2, b2, W3, b3):
    B, S = x.shape
    V, D = table.shape
    x2 = x.astype(jnp.int32)
    W1s = W1 * (1.0 / S)
    pooled = _make_pool_kernel(B, S, D)(x2, table)
    return _mlp(pooled, W1s, b1, W2, b2, W3, b3)
